# Initial kernel scaffold; baseline (speedup 1.0000x reference)
import jax
import jax.numpy as jnp
from jax.experimental import pallas as pl


def kernel(input, Adj, W1, b1, W2, b2, Wg1, bg1, Wg2, bg2):
    raise NotImplementedError("write your pallas kernel here")


# speedup vs baseline: 6.0140x; 6.0140x over previous
import functools

import jax
import jax.numpy as jnp
from jax import lax
from jax.experimental import pallas as pl
from jax.experimental.pallas import tpu as pltpu
from jax.experimental.pallas import tpu_sc as plsc

_EOS = 1e-10
_K = 10
_KP = 16
_N = 4096
_D = 128
_BR = 256
_NB = _N // _BR
_F32 = jnp.float32


def _deg_body(a_ref, d_ref, dinv_ref):
    s = jnp.sum(a_ref[...], axis=1, keepdims=True)
    d_ref[...] = s
    dinv_ref[...] = 1.0 / (jnp.sqrt(s) + _EOS)


def _prop_body(a_ref, x_ref, w_ref, b_ref, dinv_ref, dinvt_ref, o_ref, ts_ref,
               *, mode):
    i = pl.program_id(0)

    @pl.when(i == 0)
    def _():
        ts_ref[...] = (
            jnp.dot(x_ref[...], w_ref[...], preferred_element_type=_F32)
            + b_ref[...])

    na = (dinv_ref[pl.ds(i * _BR, _BR), :] * a_ref[...]) * dinvt_ref[...]
    e = jnp.dot(na, ts_ref[...], preferred_element_type=_F32)
    if mode == "tanh":
        o_ref[...] = jnp.tanh(e)
    else:
        n = jnp.sqrt(jnp.sum(e * e, axis=1, keepdims=True))
        o_ref[...] = e / jnp.maximum(n, 1e-12)


def _sim_block(emb_ref, i):
    eb = emb_ref[pl.ds(i * _BR, _BR), :]
    em = emb_ref[...]
    h = _D // 2
    s1 = jax.lax.dot_general(
        eb[:, :h], em[:, :h], (((1,), (1,)), ((), ())),
        preferred_element_type=_F32)
    s2 = jax.lax.dot_general(
        eb[:, h:], em[:, h:], (((1,), (1,)), ((), ())),
        preferred_element_type=_F32)
    return s1 + s2


def _topk_body(emb_ref, idx_ref, val_ref):
    i = pl.program_id(0)
    work = _sim_block(emb_ref, i)
    cols = jax.lax.broadcasted_iota(jnp.int32, (_BR, _N), 1)
    idxs, vals = [], []
    for _ in range(_K):
        m = jnp.max(work, axis=1, keepdims=True)
        first = jnp.min(jnp.where(work == m, cols, _N), axis=1, keepdims=True)
        idxs.append(first)
        vals.append(m)
        work = jnp.where(cols == first, -jnp.inf, work)
    pad_i = jnp.full((_BR, _KP - _K), -1, jnp.int32)
    pad_v = jnp.zeros((_BR, _KP - _K), _F32)
    idx_ref[...] = jnp.concatenate(idxs + [pad_i], axis=1)
    val_ref[...] = jnp.concatenate(vals + [pad_v], axis=1)


def _mask_block(idx_ref, idxt_ref, i, cols, rows):
    ib = idx_ref[pl.ds(i * _BR, _BR), :]
    m = ib[:, 0:1] == cols
    for k in range(_K):
        if k:
            m = m | (ib[:, k:k + 1] == cols)
        m = m | (idxt_ref[k:k + 1, :] == rows)
    return m


_NW = 16
_RW = _N // _NW
_EW = _RW * _KP
_ER = _N * _KP // 128
_ECH = _EW // 128


def _srow_sc_body(idxc1_hbm, idxc2_hbm, vals1_hbm, vals2_hbm, out_hbm,
                  idxf_v, idxs_v, vals1_v, vals2_v, own_v, tmp_v, zero_v,
                  shared):
    w = lax.axis_index("s")
    lanes = lax.iota(jnp.int32, 16)

    pltpu.sync_copy(idxc1_hbm, idxf_v)
    pltpu.sync_copy(idxc2_hbm.at[pl.ds(w * _ECH, _ECH)], idxs_v)
    pltpu.sync_copy(vals1_hbm.at[pl.ds(w * _EW, _EW)], vals1_v)
    pltpu.sync_copy(vals2_hbm.at[pl.ds(w * _ECH, _ECH)], vals2_v)

    @pl.when(w == 0)
    def _():
        def zbody(t, c):
            zero_v[pl.ds(t * 16, 16)] = jnp.zeros((16,), _F32)
            return c
        lax.fori_loop(0, _N // 16, zbody, 0)
        pltpu.sync_copy(zero_v, shared)

    plsc.subcore_barrier()

    def sbody(c, carry):
        pltpu.sync_copy(vals2_v.at[c], shared.at[idxs_v.at[c]], add=True)
        return carry
    lax.fori_loop(0, _ECH, sbody, 0)

    def gbody(g, carry):
        ivec = w * _RW + g * 16 + lanes
        acc = jnp.zeros((16,), _F32)
        for k in range(_K):
            lf = g * 256 + lanes * 16 + k
            gf = w * _EW + lf
            vi = plsc.load_gather(vals1_v, [lf])
            jc = plsc.load_gather(idxf_v, [gf])
            m = None
            for kk in range(_K):
                t = plsc.load_gather(idxf_v, [jc * _KP + kk])
                e = t == ivec
                m = e if m is None else (m | e)
            acc = acc + jnp.where(m, 0.0, vi)
        own_v[pl.ds(g * 16, 16)] = acc
        return carry
    lax.fori_loop(0, _RW // 16, gbody, 0)

    plsc.subcore_barrier()

    pltpu.sync_copy(shared.at[pl.ds(w * _RW, _RW)], tmp_v)

    def fbody(t, carry):
        s = t * 16
        tmp_v[pl.ds(s, 16)] = tmp_v[pl.ds(s, 16)] + own_v[pl.ds(s, 16)]
        return carry
    lax.fori_loop(0, _RW // 16, fbody, 0)
    pltpu.sync_copy(tmp_v, out_hbm.at[pl.ds(w * _RW, _RW)])


def _srow_sc(idx, vals):
    idxc = jnp.maximum(idx, 0)
    mesh = plsc.VectorSubcoreMesh(
        core_axis_name="c", subcore_axis_name="s", num_cores=1)
    return pl.kernel(
        _srow_sc_body,
        out_type=jax.ShapeDtypeStruct((_N,), _F32),
        mesh=mesh,
        compiler_params=pltpu.CompilerParams(needs_layout_passes=False),
        scratch_types=[
            pltpu.VMEM((_N * _KP,), jnp.int32),
            pltpu.VMEM((_ECH, 128), jnp.int32),
            pltpu.VMEM((_EW,), _F32),
            pltpu.VMEM((_ECH, 128), _F32),
            pltpu.VMEM((_RW,), _F32),
            pltpu.VMEM((_RW,), _F32),
            pltpu.VMEM((_N,), _F32),
            pltpu.VMEM_SHARED((_N,), _F32),
        ],
    )(idxc.reshape(_N * _KP), idxc.reshape(_ER, 128),
      vals.reshape(_N * _KP), vals.reshape(_ER, 128))


def _dninv_body(d_ref, srow_ref, dninv_ref):
    dn = d_ref[...] + srow_ref[...]
    dninv_ref[...] = 1.0 / (jnp.sqrt(dn) + _EOS)


def _apply_body(a_ref, emb_ref, idx_ref, idxt_ref, x_ref, w_ref, b_ref,
                dninv_ref, dninvt_ref, o_ref, ys_ref, *, relu):
    i = pl.program_id(0)

    @pl.when(i == 0)
    def _():
        ys_ref[...] = (
            jnp.dot(x_ref[...], w_ref[...], preferred_element_type=_F32)
            + b_ref[...])

    sim = _sim_block(emb_ref, i)
    cols = jax.lax.broadcasted_iota(jnp.int32, (_BR, _N), 1)
    rows = jax.lax.broadcasted_iota(jnp.int32, (_BR, _N), 0) + i * _BR
    mask = _mask_block(idx_ref, idxt_ref, i, cols, rows)
    ahat = a_ref[...] + jnp.where(mask, sim, 0.0)
    ahat = (dninv_ref[pl.ds(i * _BR, _BR), :] * ahat) * dninvt_ref[...]
    z = jnp.dot(ahat, ys_ref[...], preferred_element_type=_F32)
    if relu:
        z = jnp.maximum(z, 0.0)
    o_ref[...] = z


def _full(shape):
    nd = len(shape)
    return pl.BlockSpec(shape, lambda i, _nd=nd: (0,) * _nd)


_A_SPEC = pl.BlockSpec((_BR, _N), lambda i: (i, 0))
_ROW_SPEC = pl.BlockSpec((_BR, _D), lambda i: (i, 0))
_COL1_SPEC = pl.BlockSpec((_BR, 1), lambda i: (i, 0))


def kernel(input, Adj, W1, b1, W2, b2, Wg1, bg1, Wg2, bg2):
    x = input[0]
    A = Adj[0]
    b1r, b2r = b1[None, :], b2[None, :]
    bg1r, bg2r = bg1[None, :], bg2[None, :]

    d, dinv = pl.pallas_call(
        _deg_body,
        grid=(_NB,),
        in_specs=[_A_SPEC],
        out_specs=[_COL1_SPEC, _COL1_SPEC],
        out_shape=[jax.ShapeDtypeStruct((_N, 1), _F32)] * 2,
    )(A)

    dinvt = dinv.reshape(1, _N)
    prop = lambda mode, xin, w, b: pl.pallas_call(
        functools.partial(_prop_body, mode=mode),
        grid=(_NB,),
        in_specs=[_A_SPEC, _full((_N, _D)), _full((_D, _D)),
                  _full((1, _D)), _full((_N, 1)), _full((1, _N))],
        out_specs=_ROW_SPEC,
        out_shape=jax.ShapeDtypeStruct((_N, _D), _F32),
        scratch_shapes=[pltpu.VMEM((_N, _D), _F32)],
    )(A, xin, w, b, dinv, dinvt)

    emb1 = prop("tanh", x, Wg1, bg1r)
    emb = prop("norm", emb1, Wg2, bg2r)

    idx, vals = pl.pallas_call(
        _topk_body,
        grid=(_NB,),
        in_specs=[_full((_N, _D))],
        out_specs=[pl.BlockSpec((_BR, _KP), lambda i: (i, 0))] * 2,
        out_shape=[jax.ShapeDtypeStruct((_N, _KP), jnp.int32),
                   jax.ShapeDtypeStruct((_N, _KP), _F32)],
    )(emb)
    idxt = idx.T.copy()

    srow = _srow_sc(idx, vals)
    dninv = pl.pallas_call(
        _dninv_body,
        out_shape=jax.ShapeDtypeStruct((_N, 1), _F32),
    )(d, srow[:, None])

    dninvt = dninv.reshape(1, _N)
    apply = lambda relu, xin, w, b: pl.pallas_call(
        functools.partial(_apply_body, relu=relu),
        grid=(_NB,),
        in_specs=[_A_SPEC, _full((_N, _D)), _full((_N, _KP)),
                  _full((_KP, _N)), _full((_N, _D)), _full((_D, _D)),
                  _full((1, _D)), _full((_N, 1)), _full((1, _N))],
        out_specs=_ROW_SPEC,
        out_shape=jax.ShapeDtypeStruct((_N, _D), _F32),
        scratch_shapes=[pltpu.VMEM((_N, _D), _F32)],
    )(A, emb, idx, idxt, xin, w, b, dninv, dninvt)

    h = apply(True, x, W1, b1r)
    out = apply(False, h, W2, b2r)
    return out[None]


# speedup vs baseline: 9.2515x; 1.5383x over previous
import functools

import jax
import jax.numpy as jnp
from jax import lax
from jax.experimental import pallas as pl
from jax.experimental.pallas import tpu as pltpu
from jax.experimental.pallas import tpu_sc as plsc

_EOS = 1e-10
_K = 10
_KP = 16
_N = 4096
_D = 128
_BR = 256
_NB = _N // _BR
_F32 = jnp.float32


def _deg_body(a_ref, d_ref, dinv_ref):
    s = jnp.sum(a_ref[...], axis=1, keepdims=True)
    d_ref[...] = s
    dinv_ref[...] = 1.0 / (jnp.sqrt(s) + _EOS)


def _prop_body(a_ref, x_ref, w_ref, b_ref, dinv_ref, dinvt_ref, o_ref, ts_ref,
               *, mode):
    i = pl.program_id(0)

    @pl.when(i == 0)
    def _():
        ts_ref[...] = (
            jnp.dot(x_ref[...], w_ref[...], preferred_element_type=_F32)
            + b_ref[...])

    na = (dinv_ref[pl.ds(i * _BR, _BR), :] * a_ref[...]) * dinvt_ref[...]
    e = jnp.dot(na, ts_ref[...], preferred_element_type=_F32)
    if mode == "tanh":
        o_ref[...] = jnp.tanh(e)
    else:
        n = jnp.sqrt(jnp.sum(e * e, axis=1, keepdims=True))
        o_ref[...] = e / jnp.maximum(n, 1e-12)


def _sim_block(emb_ref, i):
    eb = emb_ref[pl.ds(i * _BR, _BR), :]
    em = emb_ref[...]
    h = _D // 2
    s1 = jax.lax.dot_general(
        eb[:, :h], em[:, :h], (((1,), (1,)), ((), ())),
        preferred_element_type=_F32)
    s2 = jax.lax.dot_general(
        eb[:, h:], em[:, h:], (((1,), (1,)), ((), ())),
        preferred_element_type=_F32)
    return s1 + s2


def _topk_body(emb_ref, idx_ref, val_ref):
    i = pl.program_id(0)
    work = _sim_block(emb_ref, i)
    cols = jax.lax.broadcasted_iota(jnp.int32, (_BR, _N), 1)
    idxs, vals = [], []
    for _ in range(_K):
        m = jnp.max(work, axis=1, keepdims=True)
        first = jnp.min(jnp.where(work == m, cols, _N), axis=1, keepdims=True)
        idxs.append(first)
        vals.append(m)
        work = jnp.where(cols == first, -jnp.inf, work)
    pad_i = jnp.full((_BR, _KP - _K), -1, jnp.int32)
    pad_v = jnp.zeros((_BR, _KP - _K), _F32)
    idx_ref[...] = jnp.concatenate(idxs + [pad_i], axis=1)
    val_ref[...] = jnp.concatenate(vals + [pad_v], axis=1)


def _mask_block(idx_ref, idxt_ref, i, cols, rows):
    ib = idx_ref[pl.ds(i * _BR, _BR), :]
    m = ib[:, 0:1] == cols
    for k in range(_K):
        if k:
            m = m | (ib[:, k:k + 1] == cols)
        m = m | (idxt_ref[k:k + 1, :] == rows)
    return m


_NW = 16
_RW = _N // _NW
_EW = _RW * _KP
_ER = _N * _KP // 128
_ECH = _EW // 128


def _srow_sc_body(idxc1_hbm, idxc2_hbm, vals1_hbm, vals2_hbm, out_hbm,
                  idxf_v, idxs_v, vals1_v, vals2_v, own_v, tmp_v, zero_v,
                  shared):
    w = lax.axis_index("s")
    lanes = lax.iota(jnp.int32, 16)

    pltpu.sync_copy(idxc1_hbm, idxf_v)
    pltpu.sync_copy(idxc2_hbm.at[pl.ds(w * _ECH, _ECH)], idxs_v)
    pltpu.sync_copy(vals1_hbm.at[pl.ds(w * _EW, _EW)], vals1_v)
    pltpu.sync_copy(vals2_hbm.at[pl.ds(w * _ECH, _ECH)], vals2_v)

    @pl.when(w == 0)
    def _():
        def zbody(t, c):
            zero_v[pl.ds(t * 16, 16)] = jnp.zeros((16,), _F32)
            return c
        lax.fori_loop(0, _N // 16, zbody, 0)
        pltpu.sync_copy(zero_v, shared)

    plsc.subcore_barrier()

    def sbody(c, carry):
        pltpu.sync_copy(vals2_v.at[c], shared.at[idxs_v.at[c]], add=True)
        return carry
    lax.fori_loop(0, _ECH, sbody, 0)

    def gbody(g, carry):
        ivec = w * _RW + g * 16 + lanes
        acc = jnp.zeros((16,), _F32)
        for k in range(_K):
            lf = g * 256 + lanes * 16 + k
            gf = w * _EW + lf
            vi = plsc.load_gather(vals1_v, [lf])
            jc = plsc.load_gather(idxf_v, [gf])
            m = None
            for kk in range(_K):
                t = plsc.load_gather(idxf_v, [jc * _KP + kk])
                e = t == ivec
                m = e if m is None else (m | e)
            acc = acc + jnp.where(m, 0.0, vi)
        own_v[pl.ds(g * 16, 16)] = acc
        return carry
    lax.fori_loop(0, _RW // 16, gbody, 0)

    plsc.subcore_barrier()

    pltpu.sync_copy(shared.at[pl.ds(w * _RW, _RW)], tmp_v)

    def fbody(t, carry):
        s = t * 16
        tmp_v[pl.ds(s, 16)] = tmp_v[pl.ds(s, 16)] + own_v[pl.ds(s, 16)]
        return carry
    lax.fori_loop(0, _RW // 16, fbody, 0)
    pltpu.sync_copy(tmp_v, out_hbm.at[pl.ds(w * _RW, _RW)])


def _srow_sc(idx, vals):
    idxc = jnp.maximum(idx, 0)
    mesh = plsc.VectorSubcoreMesh(
        core_axis_name="c", subcore_axis_name="s", num_cores=1)
    return pl.kernel(
        _srow_sc_body,
        out_type=jax.ShapeDtypeStruct((_N,), _F32),
        mesh=mesh,
        compiler_params=pltpu.CompilerParams(needs_layout_passes=False),
        scratch_types=[
            pltpu.VMEM((_N * _KP,), jnp.int32),
            pltpu.VMEM((_ECH, 128), jnp.int32),
            pltpu.VMEM((_EW,), _F32),
            pltpu.VMEM((_ECH, 128), _F32),
            pltpu.VMEM((_RW,), _F32),
            pltpu.VMEM((_RW,), _F32),
            pltpu.VMEM((_N,), _F32),
            pltpu.VMEM_SHARED((_N,), _F32),
        ],
    )(idxc.reshape(_N * _KP), idxc.reshape(_ER, 128),
      vals.reshape(_N * _KP), vals.reshape(_ER, 128))


def _dninv_body(d_ref, srow_ref, dninv_ref):
    dn = d_ref[...] + srow_ref[...]
    dninv_ref[...] = 1.0 / (jnp.sqrt(dn) + _EOS)


def _gcn1_body(a_ref, emb_ref, idx_ref, idxt_ref, x_ref, w_ref, b_ref,
               dninv_ref, dninvt_ref, o_ref, ah_ref, ys_ref):
    i = pl.program_id(0)

    @pl.when(i == 0)
    def _():
        ys_ref[...] = (
            jnp.dot(x_ref[...], w_ref[...], preferred_element_type=_F32)
            + b_ref[...])

    sim = _sim_block(emb_ref, i)
    cols = jax.lax.broadcasted_iota(jnp.int32, (_BR, _N), 1)
    rows = jax.lax.broadcasted_iota(jnp.int32, (_BR, _N), 0) + i * _BR
    mask = _mask_block(idx_ref, idxt_ref, i, cols, rows)
    ahat = a_ref[...] + jnp.where(mask, sim, 0.0)
    ahat = (dninv_ref[pl.ds(i * _BR, _BR), :] * ahat) * dninvt_ref[...]
    ah_ref[...] = ahat
    z = jnp.dot(ahat, ys_ref[...], preferred_element_type=_F32)
    o_ref[...] = jnp.maximum(z, 0.0)


def _gcn2_body(ah_ref, x_ref, w_ref, b_ref, o_ref, ys_ref):
    i = pl.program_id(0)

    @pl.when(i == 0)
    def _():
        ys_ref[...] = (
            jnp.dot(x_ref[...], w_ref[...], preferred_element_type=_F32)
            + b_ref[...])

    o_ref[...] = jnp.dot(ah_ref[...], ys_ref[...],
                         preferred_element_type=_F32)


def _full(shape):
    nd = len(shape)
    return pl.BlockSpec(shape, lambda i, _nd=nd: (0,) * _nd)


_A_SPEC = pl.BlockSpec((_BR, _N), lambda i: (i, 0))
_ROW_SPEC = pl.BlockSpec((_BR, _D), lambda i: (i, 0))
_COL1_SPEC = pl.BlockSpec((_BR, 1), lambda i: (i, 0))


def kernel(input, Adj, W1, b1, W2, b2, Wg1, bg1, Wg2, bg2):
    x = input[0]
    A = Adj[0]
    b1r, b2r = b1[None, :], b2[None, :]
    bg1r, bg2r = bg1[None, :], bg2[None, :]

    d, dinv = pl.pallas_call(
        _deg_body,
        grid=(_NB,),
        in_specs=[_A_SPEC],
        out_specs=[_COL1_SPEC, _COL1_SPEC],
        out_shape=[jax.ShapeDtypeStruct((_N, 1), _F32)] * 2,
    )(A)

    dinvt = dinv.reshape(1, _N)
    prop = lambda mode, xin, w, b: pl.pallas_call(
        functools.partial(_prop_body, mode=mode),
        grid=(_NB,),
        in_specs=[_A_SPEC, _full((_N, _D)), _full((_D, _D)),
                  _full((1, _D)), _full((_N, 1)), _full((1, _N))],
        out_specs=_ROW_SPEC,
        out_shape=jax.ShapeDtypeStruct((_N, _D), _F32),
        scratch_shapes=[pltpu.VMEM((_N, _D), _F32)],
    )(A, xin, w, b, dinv, dinvt)

    emb1 = prop("tanh", x, Wg1, bg1r)
    emb = prop("norm", emb1, Wg2, bg2r)

    idx, vals = pl.pallas_call(
        _topk_body,
        grid=(_NB,),
        in_specs=[_full((_N, _D))],
        out_specs=[pl.BlockSpec((_BR, _KP), lambda i: (i, 0))] * 2,
        out_shape=[jax.ShapeDtypeStruct((_N, _KP), jnp.int32),
                   jax.ShapeDtypeStruct((_N, _KP), _F32)],
    )(emb)
    idxt = idx.T.copy()

    srow = _srow_sc(idx, vals)
    dninv = pl.pallas_call(
        _dninv_body,
        out_shape=jax.ShapeDtypeStruct((_N, 1), _F32),
    )(d, srow[:, None])

    dninvt = dninv.reshape(1, _N)
    h, ah = pl.pallas_call(
        _gcn1_body,
        grid=(_NB,),
        in_specs=[_A_SPEC, _full((_N, _D)), _full((_N, _KP)),
                  _full((_KP, _N)), _full((_N, _D)), _full((_D, _D)),
                  _full((1, _D)), _full((_N, 1)), _full((1, _N))],
        out_specs=[_ROW_SPEC, _A_SPEC],
        out_shape=[jax.ShapeDtypeStruct((_N, _D), _F32),
                   jax.ShapeDtypeStruct((_N, _N), _F32)],
        scratch_shapes=[pltpu.VMEM((_N, _D), _F32)],
    )(A, emb, idx, idxt, x, W1, b1r, dninv, dninvt)

    out = pl.pallas_call(
        _gcn2_body,
        grid=(_NB,),
        in_specs=[_A_SPEC, _full((_N, _D)), _full((_D, _D)), _full((1, _D))],
        out_specs=_ROW_SPEC,
        out_shape=jax.ShapeDtypeStruct((_N, _D), _F32),
        scratch_shapes=[pltpu.VMEM((_N, _D), _F32)],
    )(ah, h, W2, b2r)
    return out[None]
